# baseline (speedup 1.0000x reference)
import functools

import jax
import jax.numpy as jnp
from jax import lax
from jax.experimental import pallas as pl
from jax.experimental.pallas import tpu as pltpu
from jax.experimental.pallas import tpu_sc as plsc

NC = 2
NS = 16
CHUNK = 128


def _round_up(a, b):
    return (a + b - 1) // b * b


def _matmul_z(x, w):
    v_tot, inc = x.shape
    k_tot, _, outc = w.shape
    vb = 400 if v_tot % 400 == 0 else v_tot
    nv = v_tot // vb

    def body(x_ref, w_ref, o_ref):
        xb = x_ref[...].astype(jnp.bfloat16)
        for k in range(k_tot):
            o_ref[k] = jnp.dot(xb, w_ref[k].astype(jnp.bfloat16),
                               preferred_element_type=jnp.float32)

    return pl.pallas_call(
        body,
        grid=(nv,),
        in_specs=[
            pl.BlockSpec((vb, inc), lambda v: (v, 0)),
            pl.BlockSpec((k_tot, inc, outc), lambda v: (0, 0, 0)),
        ],
        out_specs=pl.BlockSpec((k_tot, vb, outc), lambda v: (0, v, 0)),
        out_shape=jax.ShapeDtypeStruct((k_tot, v_tot, outc), jnp.float32),
    )(x, w)


def _add_partials(partials, n_vox):
    _, _, outc = partials.shape
    vb = 2000 if n_vox % 2000 == 0 else n_vox
    nv = n_vox // vb

    def body(a_ref, b_ref, o_ref):
        o_ref[...] = a_ref[0] + b_ref[0]

    return pl.pallas_call(
        body,
        grid=(nv,),
        in_specs=[
            pl.BlockSpec((1, vb, outc), lambda v: (0, v, 0)),
            pl.BlockSpec((1, vb, outc), lambda v: (1, v, 0)),
        ],
        out_specs=pl.BlockSpec((vb, outc), lambda v: (v, 0)),
        out_shape=jax.ShapeDtypeStruct((n_vox, outc), jnp.float32),
    )(partials, partials)


def _make_sc_scatter(n_vox, outc, acc_rows, cpw0, cpw1, n_rows):
    rows_per_sub = acc_rows // NS
    cpp_max = max(cpw0, cpw1) // 2
    mesh = plsc.VectorSubcoreMesh(core_axis_name="c", subcore_axis_name="s")

    @functools.partial(
        pl.kernel,
        mesh=mesh,
        out_type=jax.ShapeDtypeStruct((NC, acc_rows, outc), jnp.float32),
        scratch_types=[
            pltpu.VMEM((cpp_max, CHUNK), jnp.int32),
            pltpu.VMEM((cpp_max, CHUNK), jnp.int32),
            pltpu.VMEM((CHUNK, outc), jnp.float32),
            pltpu.VMEM((CHUNK, outc), jnp.float32),
            pltpu.VMEM_SHARED((acc_rows, outc), jnp.float32),
            pltpu.SemaphoreType.DMA,
            pltpu.SemaphoreType.DMA,
        ],
    )
    def sc_scatter(z_hbm, gidx_hbm, sidx_hbm, out_hbm,
                   gidx_v, sidx_v, rows_a, rows_b, acc, sem_a, sem_b):
        c = lax.axis_index("c")
        s = lax.axis_index("s")
        zero16 = jnp.zeros((16,), jnp.float32)

        with jax.named_scope("acc_zero"):
            def zrow(i, carry):
                for l in range(outc // 16):
                    rows_a[i, pl.ds(l * 16, 16)] = zero16
                return carry

            lax.fori_loop(0, CHUNK, zrow, 0)
            for off in range(0, rows_per_sub, CHUNK):
                sz = min(CHUNK, rows_per_sub - off)
                pltpu.sync_copy(
                    rows_a.at[pl.ds(0, sz)],
                    acc.at[pl.ds(s * rows_per_sub + off, sz)])
            plsc.subcore_barrier()

        def run_range(first_row, cpw):
            cpp = cpw // 2
            for p in range(2):
                base = first_row + p * cpp
                pltpu.sync_copy(gidx_hbm.at[pl.ds(base, cpp)],
                                gidx_v.at[pl.ds(0, cpp)])
                pltpu.sync_copy(sidx_hbm.at[pl.ds(base, cpp)],
                                sidx_v.at[pl.ds(0, cpp)])
                pltpu.async_copy(z_hbm.at[gidx_v.at[0]], rows_a, sem_a)

                def body(i, carry):
                    j = 2 * i
                    pltpu.async_copy(z_hbm.at[gidx_v.at[j + 1]], rows_b,
                                     sem_b)
                    pltpu.make_async_copy(z_hbm.at[gidx_v.at[j]], rows_a,
                                          sem_a).wait()
                    pltpu.sync_copy(rows_a, acc.at[sidx_v.at[j]], add=True)

                    @pl.when(j + 2 < cpp)
                    def _():
                        pltpu.async_copy(z_hbm.at[gidx_v.at[j + 2]], rows_a,
                                         sem_a)

                    pltpu.make_async_copy(z_hbm.at[gidx_v.at[j + 1]], rows_b,
                                          sem_b).wait()
                    pltpu.sync_copy(rows_b, acc.at[sidx_v.at[j + 1]],
                                    add=True)
                    return carry

                lax.fori_loop(0, cpp // 2, body, 0)

        with jax.named_scope("chunk_loop"):
            @pl.when(c == 0)
            def _():
                run_range(s * cpw0, cpw0)

            @pl.when(c == 1)
            def _():
                run_range(NS * cpw0 + s * cpw1, cpw1)

            plsc.subcore_barrier()
        with jax.named_scope("writeback"):
            pltpu.sync_copy(acc.at[pl.ds(s * rows_per_sub, rows_per_sub)],
                            out_hbm.at[c, pl.ds(s * rows_per_sub,
                                                rows_per_sub)])

    return sc_scatter


def kernel(x, nbr_in, nbr_out, kernel):
    n_vox, inc = x.shape
    k_tot, e_tot = nbr_in.shape
    outc = kernel.shape[-1]

    z = _matmul_z(x, kernel).reshape(k_tot * n_vox, outc)

    n_pairs = k_tot * e_tot
    pairs_pad = _round_up(n_pairs, NC * NS * CHUNK)
    n_rows = pairs_pad // CHUNK
    per_lane = n_rows // NS
    cpw0 = (per_lane * 28 // 40) // 16 * 16
    cpw1 = per_lane - cpw0
    acc_rows = _round_up(n_vox + 1, NS * 8)

    koff = (jnp.arange(k_tot, dtype=jnp.int32) * n_vox)[:, None]
    gflat = (nbr_in.astype(jnp.int32) + koff).reshape(-1)
    gidx = jnp.concatenate(
        [gflat, jnp.zeros((pairs_pad - n_pairs,), jnp.int32)]
    ).reshape(n_rows, CHUNK)
    sflat = nbr_out.astype(jnp.int32).reshape(-1)
    n_dump = acc_rows - n_vox
    pad_dst = n_vox + jnp.arange(pairs_pad - n_pairs, dtype=jnp.int32) % n_dump
    sidx = jnp.concatenate([sflat, pad_dst]).reshape(n_rows, CHUNK)

    sc = _make_sc_scatter(n_vox, outc, acc_rows, cpw0, cpw1, n_rows)
    partials = sc(z, gidx, sidx)

    return _add_partials(partials, n_vox)


# speedup vs baseline: 1.1389x; 1.1389x over previous
import functools

import jax
import jax.numpy as jnp
from jax import lax
from jax.experimental import pallas as pl
from jax.experimental.pallas import tpu as pltpu
from jax.experimental.pallas import tpu_sc as plsc

NC = 2
NS = 16
CHUNK = 128


def _round_up(a, b):
    return (a + b - 1) // b * b


def _matmul_z(x, w):
    v_tot, inc = x.shape
    k_tot, _, outc = w.shape
    vb = 400 if v_tot % 400 == 0 else v_tot
    nv = v_tot // vb

    def body(x_ref, w_ref, o_ref):
        xb = x_ref[...].astype(jnp.bfloat16)
        for k in range(k_tot):
            o_ref[k] = jnp.dot(xb, w_ref[k].astype(jnp.bfloat16),
                               preferred_element_type=jnp.float32)

    return pl.pallas_call(
        body,
        grid=(nv,),
        in_specs=[
            pl.BlockSpec((vb, inc), lambda v: (v, 0)),
            pl.BlockSpec((k_tot, inc, outc), lambda v: (0, 0, 0)),
        ],
        out_specs=pl.BlockSpec((k_tot, vb, outc), lambda v: (0, v, 0)),
        out_shape=jax.ShapeDtypeStruct((k_tot, v_tot, outc), jnp.float32),
    )(x, w)


def _add_partials(p0, p1):
    v_tot, outc = p0.shape
    vb = 2000 if v_tot % 2000 == 0 else v_tot
    nv = v_tot // vb

    def body(a_ref, b_ref, o_ref):
        o_ref[...] = a_ref[...] + b_ref[...]

    spec = pl.BlockSpec((vb, outc), lambda v: (v, 0))
    return pl.pallas_call(
        body,
        grid=(nv,),
        in_specs=[spec, spec],
        out_specs=spec,
        out_shape=jax.ShapeDtypeStruct((v_tot, outc), jnp.float32),
    )(p0, p1)


def _make_sc_scatter(n_vox, outc, acc_rows, cpw0, cpw1, n_rows):
    rows_per_sub = acc_rows // NS
    cpp_max = max(cpw0, cpw1) // 2
    mesh = plsc.VectorSubcoreMesh(core_axis_name="c", subcore_axis_name="s")

    @functools.partial(
        pl.kernel,
        mesh=mesh,
        out_type=jax.ShapeDtypeStruct((NC, acc_rows, outc), jnp.float32),
        scratch_types=[
            pltpu.VMEM((cpp_max, CHUNK), jnp.int32),
            pltpu.VMEM((cpp_max, CHUNK), jnp.int32),
            pltpu.VMEM((CHUNK, outc), jnp.float32),
            pltpu.VMEM((CHUNK, outc), jnp.float32),
            pltpu.VMEM_SHARED((acc_rows, outc), jnp.float32),
            pltpu.SemaphoreType.DMA,
            pltpu.SemaphoreType.DMA,
        ],
    )
    def sc_scatter(z_hbm, gidx_hbm, sidx_hbm, out_hbm,
                   gidx_v, sidx_v, rows_a, rows_b, acc, sem_a, sem_b):
        c = lax.axis_index("c")
        s = lax.axis_index("s")
        zero16 = jnp.zeros((16,), jnp.float32)

        with jax.named_scope("acc_zero"):
            def zrow(i, carry):
                for l in range(outc // 16):
                    rows_a[i, pl.ds(l * 16, 16)] = zero16
                return carry

            lax.fori_loop(0, CHUNK, zrow, 0)
            for off in range(0, rows_per_sub, CHUNK):
                sz = min(CHUNK, rows_per_sub - off)
                pltpu.sync_copy(
                    rows_a.at[pl.ds(0, sz)],
                    acc.at[pl.ds(s * rows_per_sub + off, sz)])
            plsc.subcore_barrier()

        def run_range(first_row, cpw):
            cpp = cpw // 2
            for p in range(2):
                base = first_row + p * cpp
                pltpu.sync_copy(gidx_hbm.at[pl.ds(base, cpp)],
                                gidx_v.at[pl.ds(0, cpp)])
                pltpu.sync_copy(sidx_hbm.at[pl.ds(base, cpp)],
                                sidx_v.at[pl.ds(0, cpp)])
                pltpu.async_copy(z_hbm.at[gidx_v.at[0]], rows_a, sem_a)

                def body(i, carry):
                    j = 2 * i
                    pltpu.async_copy(z_hbm.at[gidx_v.at[j + 1]], rows_b,
                                     sem_b)
                    pltpu.make_async_copy(z_hbm.at[gidx_v.at[j]], rows_a,
                                          sem_a).wait()
                    pltpu.sync_copy(rows_a, acc.at[sidx_v.at[j]], add=True)

                    @pl.when(j + 2 < cpp)
                    def _():
                        pltpu.async_copy(z_hbm.at[gidx_v.at[j + 2]], rows_a,
                                         sem_a)

                    pltpu.make_async_copy(z_hbm.at[gidx_v.at[j + 1]], rows_b,
                                          sem_b).wait()
                    pltpu.sync_copy(rows_b, acc.at[sidx_v.at[j + 1]],
                                    add=True)
                    return carry

                lax.fori_loop(0, cpp // 2, body, 0)

        with jax.named_scope("chunk_loop"):
            @pl.when(c == 0)
            def _():
                run_range(s * cpw0, cpw0)

            @pl.when(c == 1)
            def _():
                run_range(NS * cpw0 + s * cpw1, cpw1)

            plsc.subcore_barrier()
        with jax.named_scope("writeback"):
            pltpu.sync_copy(acc.at[pl.ds(s * rows_per_sub, rows_per_sub)],
                            out_hbm.at[c, pl.ds(s * rows_per_sub,
                                                rows_per_sub)])

    return sc_scatter


def kernel(x, nbr_in, nbr_out, kernel):
    n_vox, inc = x.shape
    k_tot, e_tot = nbr_in.shape
    outc = kernel.shape[-1]

    z = _matmul_z(x, kernel).reshape(k_tot * n_vox, outc)

    n_pairs = k_tot * e_tot
    pairs_pad = _round_up(n_pairs, NC * NS * CHUNK)
    n_rows = pairs_pad // CHUNK
    per_lane = n_rows // NS
    cpw0 = (per_lane * 28 // 40) // 16 * 16
    cpw1 = per_lane - cpw0
    acc_rows = _round_up(n_vox + 1, NS * 8)

    koff = (jnp.arange(k_tot, dtype=jnp.int32) * n_vox)[:, None]
    gflat = (nbr_in.astype(jnp.int32) + koff).reshape(-1)
    gidx = jnp.concatenate(
        [gflat, jnp.zeros((pairs_pad - n_pairs,), jnp.int32)]
    ).reshape(n_rows, CHUNK)
    sflat = nbr_out.astype(jnp.int32).reshape(-1)
    n_dump = acc_rows - n_vox
    pad_dst = n_vox + jnp.arange(pairs_pad - n_pairs, dtype=jnp.int32) % n_dump
    sidx = jnp.concatenate([sflat, pad_dst]).reshape(n_rows, CHUNK)

    sc = _make_sc_scatter(n_vox, outc, acc_rows, cpw0, cpw1, n_rows)
    partials = sc(z, gidx, sidx)

    return _add_partials(partials[0, :n_vox], partials[1, :n_vox])
